# baseline (speedup 1.0000x reference)
import jax
import jax.numpy as jnp
from jax.experimental import pallas as pl
from jax.experimental.pallas import tpu as pltpu

_K = 4096
_ONE_B16 = 0x3F80
_B = 16
_ROWS = 2048
_LANES = 128
_G = 4
_STEPS = _B // _G


def _bce_topk_kernel(pred_ref, tgt_ref, out_ref, pbf_ref, lo_ref):
    i = pl.program_id(0)

    @pl.when(i < _STEPS)
    def _prep():
        p = jnp.abs(pred_ref[...] + (tgt_ref[...].astype(jnp.float32) - 1.0))
        pbf_ref[pl.ds(i * _G, _G), :, :] = p.astype(jnp.bfloat16)

    @pl.when(i == _STEPS)
    def _search():
        ph = jax.lax.bitcast_convert_type(pbf_ref[...], jnp.int16)
        lo = jnp.zeros((_B, 1, 1), jnp.int32)
        hi = jnp.full((_B, 1, 1), _ONE_B16, jnp.int32)

        def body(_, carry):
            lo, hi = carry
            mid = lo + (hi - lo) // 2
            x = (ph <= mid.astype(jnp.int16)).astype(jnp.int16)
            for _ in range(7):
                h = x.shape[1] // 2
                x = x[:, :h, :] + x[:, h:, :]
            cnt = jnp.sum(x.astype(jnp.int32), axis=(1, 2), keepdims=True)
            take = cnt >= _K
            return (jnp.where(take, lo, mid + 1),
                    jnp.where(take, mid, hi))

        lo, _ = jax.lax.fori_loop(0, 14, body, (lo, hi))
        lo_ref[...] = lo

    @pl.when(i >= _STEPS)
    def _finalize():
        r = (i - _STEPS) * _G
        k16 = lo_ref[pl.ds(r, _G), :, :].astype(jnp.int16)
        pbf = pbf_ref[pl.ds(r, _G), :, :]
        below = jax.lax.bitcast_convert_type(pbf, jnp.int16) < k16
        losses = -jnp.log(pbf.astype(jnp.float32))
        c_lt = jnp.sum(below.astype(jnp.int32), axis=(1, 2), keepdims=True)
        contrib = jnp.sum(jnp.where(below, losses, 0.0), axis=(1, 2),
                          keepdims=True)
        t = jax.lax.bitcast_convert_type(k16, jnp.bfloat16).astype(jnp.float32)
        row_s = contrib + (_K - c_lt).astype(jnp.float32) * (-jnp.log(t))
        acc = jnp.sum(row_s[:, :, 0], axis=0, keepdims=True) / float(_K * _B)
        prev = out_ref[...]
        out_ref[...] = jnp.where(i == _STEPS, 0.0, prev) + acc


@jax.jit
def kernel(pred, target):
    pred2 = pred.reshape(_B, _ROWS, _LANES)
    tgt2 = target.reshape(_B, _ROWS, _LANES)
    row_spec = pl.BlockSpec((_G, _ROWS, _LANES),
                            lambda i: (jnp.minimum(i, _STEPS - 1), 0, 0))
    out = pl.pallas_call(
        _bce_topk_kernel,
        grid=(2 * _STEPS,),
        in_specs=[row_spec, row_spec],
        out_specs=pl.BlockSpec((1, 1), lambda i: (0, 0)),
        out_shape=jax.ShapeDtypeStruct((1, 1), jnp.float32),
        scratch_shapes=[
            pltpu.VMEM((_B, _ROWS, _LANES), jnp.bfloat16),
            pltpu.VMEM((_B, 1, 1), jnp.int32),
        ],
    )(pred2, tgt2)
    return out.reshape(())


# speedup vs baseline: 1.1063x; 1.1063x over previous
import jax
import jax.numpy as jnp
from jax.experimental import pallas as pl
from jax.experimental.pallas import tpu as pltpu

_K = 4096
_ONE_B16 = 0x3F80
_B = 16
_ROWS = 2048
_LANES = 128
_G = 4
_STEPS = _B // _G


def _bce_topk_kernel(pred_ref, tgt_ref, out_ref, pbf_ref, lo_ref):
    i = pl.program_id(0)

    @pl.when(i < _STEPS)
    def _prep():
        p = jnp.abs(pred_ref[...] + (tgt_ref[...].astype(jnp.float32) - 1.0))
        pbf_ref[pl.ds(i * _G, _G), :, :] = p.astype(jnp.bfloat16)

    @pl.when(i == _STEPS)
    def _search():
        ph = jax.lax.bitcast_convert_type(pbf_ref[...], jnp.int16)

        def count_le(m):
            x = (ph <= m.astype(jnp.int16)).astype(jnp.int16)
            for _ in range(7):
                h = x.shape[1] // 2
                x = x[:, :h, :] + x[:, h:, :]
            return jnp.sum(x.astype(jnp.int32), axis=(1, 2), keepdims=True)

        a = jnp.full((_B, 1, 1), 0x3C60, jnp.int32)
        b = jnp.full((_B, 1, 1), 0x3CA0, jnp.int32)
        ca, cb = count_le(a), count_le(b)
        lo = jnp.where(ca >= _K, 0, jnp.where(cb >= _K, a + 1, b + 1))
        hi = jnp.where(ca >= _K, a, jnp.where(cb >= _K, b, _ONE_B16))

        def cond(carry):
            lo, hi = carry
            return jnp.any(lo < hi)

        def body(carry):
            lo, hi = carry
            mid = lo + (hi - lo) // 2
            take = count_le(mid) >= _K
            return (jnp.where(take, lo, mid + 1),
                    jnp.where(take, mid, hi))

        lo, _ = jax.lax.while_loop(cond, body, (lo, hi))
        lo_ref[...] = lo

    @pl.when(i >= _STEPS)
    def _finalize():
        r = (i - _STEPS) * _G
        k16 = lo_ref[pl.ds(r, _G), :, :].astype(jnp.int16)
        pbf = pbf_ref[pl.ds(r, _G), :, :]
        below = jax.lax.bitcast_convert_type(pbf, jnp.int16) < k16
        losses = -jnp.log(pbf.astype(jnp.float32))
        c_lt = jnp.sum(below.astype(jnp.int32), axis=(1, 2), keepdims=True)
        contrib = jnp.sum(jnp.where(below, losses, 0.0), axis=(1, 2),
                          keepdims=True)
        t = jax.lax.bitcast_convert_type(k16, jnp.bfloat16).astype(jnp.float32)
        row_s = contrib + (_K - c_lt).astype(jnp.float32) * (-jnp.log(t))
        acc = jnp.sum(row_s[:, :, 0], axis=0, keepdims=True) / float(_K * _B)
        prev = out_ref[...]
        out_ref[...] = jnp.where(i == _STEPS, 0.0, prev) + acc


@jax.jit
def kernel(pred, target):
    pred2 = pred.reshape(_B, _ROWS, _LANES)
    tgt2 = target.reshape(_B, _ROWS, _LANES)
    row_spec = pl.BlockSpec((_G, _ROWS, _LANES),
                            lambda i: (jnp.minimum(i, _STEPS - 1), 0, 0))
    out = pl.pallas_call(
        _bce_topk_kernel,
        grid=(2 * _STEPS,),
        in_specs=[row_spec, row_spec],
        out_specs=pl.BlockSpec((1, 1), lambda i: (0, 0)),
        out_shape=jax.ShapeDtypeStruct((1, 1), jnp.float32),
        scratch_shapes=[
            pltpu.VMEM((_B, _ROWS, _LANES), jnp.bfloat16),
            pltpu.VMEM((_B, 1, 1), jnp.int32),
        ],
    )(pred2, tgt2)
    return out.reshape(())


# speedup vs baseline: 1.1262x; 1.0180x over previous
import jax
import jax.numpy as jnp
from jax.experimental import pallas as pl
from jax.experimental.pallas import tpu as pltpu

_K = 4096
_ONE_B16 = 0x3F80
_B = 16
_ROWS = 2048
_LANES = 128
_G = 4
_STEPS = _B // _G


def _bce_topk_kernel(pred_ref, tgt_ref, out_ref, pbf_ref, lo_ref):
    i = pl.program_id(0)

    @pl.when(i < _STEPS)
    def _prep():
        p = jnp.abs(pred_ref[...] + (tgt_ref[...].astype(jnp.float32) - 1.0))
        pbf_ref[pl.ds(i * _G, _G), :, :] = p.astype(jnp.bfloat16)

    @pl.when(i == _STEPS)
    def _search():
        ph = jax.lax.bitcast_convert_type(pbf_ref[...], jnp.int16)

        def count_le(m):
            x = (ph <= m.astype(jnp.int16)).astype(jnp.int16)
            for _ in range(7):
                h = x.shape[1] // 2
                x = x[:, :h, :] + x[:, h:, :]
            return jnp.sum(x.astype(jnp.int32), axis=(1, 2), keepdims=True)

        a = jnp.full((_B, 1, 1), 0x3C60, jnp.int32)
        b = jnp.full((_B, 1, 1), 0x3CA0, jnp.int32)
        ca, cb = count_le(a), count_le(b)
        lo = jnp.where(ca >= _K, 0, jnp.where(cb >= _K, a + 1, b + 1))
        hi = jnp.where(ca >= _K, a, jnp.where(cb >= _K, b, _ONE_B16))

        def cond(carry):
            lo, hi = carry
            return jnp.any(lo < hi)

        def body(carry):
            lo, hi = carry
            mid = lo + (hi - lo) // 2
            take = count_le(mid) >= _K
            return (jnp.where(take, lo, mid + 1),
                    jnp.where(take, mid, hi))

        lo, _ = jax.lax.while_loop(cond, body, (lo, hi))
        lo_ref[...] = lo

    @pl.when(i >= _STEPS)
    def _finalize():
        r = (i - _STEPS) * _G
        k16 = lo_ref[pl.ds(r, _G), :, :].astype(jnp.int16)
        pbf = pbf_ref[pl.ds(r, _G), :, :]
        below = jax.lax.bitcast_convert_type(pbf, jnp.int16) < k16
        losses = -jnp.log(pbf.astype(jnp.float32))
        x = below.astype(jnp.int16)
        for _ in range(7):
            h = x.shape[1] // 2
            x = x[:, :h, :] + x[:, h:, :]
        c_lt = jnp.sum(x.astype(jnp.int32), axis=(1, 2), keepdims=True)
        contrib = jnp.sum(jnp.where(below, losses, 0.0), axis=(1, 2),
                          keepdims=True)
        t = jax.lax.bitcast_convert_type(k16, jnp.bfloat16).astype(jnp.float32)
        row_s = contrib + (_K - c_lt).astype(jnp.float32) * (-jnp.log(t))
        acc = jnp.sum(row_s[:, :, 0], axis=0, keepdims=True) / float(_K * _B)
        prev = out_ref[...]
        out_ref[...] = jnp.where(i == _STEPS, 0.0, prev) + acc


@jax.jit
def kernel(pred, target):
    pred2 = pred.reshape(_B, _ROWS, _LANES)
    tgt2 = target.reshape(_B, _ROWS, _LANES)
    row_spec = pl.BlockSpec((_G, _ROWS, _LANES),
                            lambda i: (jnp.minimum(i, _STEPS - 1), 0, 0))
    out = pl.pallas_call(
        _bce_topk_kernel,
        grid=(2 * _STEPS,),
        in_specs=[row_spec, row_spec],
        out_specs=pl.BlockSpec((1, 1), lambda i: (0, 0)),
        out_shape=jax.ShapeDtypeStruct((1, 1), jnp.float32),
        scratch_shapes=[
            pltpu.VMEM((_B, _ROWS, _LANES), jnp.bfloat16),
            pltpu.VMEM((_B, 1, 1), jnp.int32),
        ],
    )(pred2, tgt2)
    return out.reshape(())
